# baseline (speedup 1.0000x reference)
import functools

import jax
import jax.numpy as jnp
from jax import lax
from jax.experimental import pallas as pl
from jax.experimental.pallas import tpu as pltpu
from jax.experimental.pallas import tpu_sc as plsc

N = 10000
E = 320000
D = 128

NC, NS = 2, 16
NW = NC * NS
EPW = E // NW
K = 80
NCH = EPW // K
R = 4
NI = 8
IL = 6
GL = 2
RPT = 624
ZR = 16
CZ = 2000
UNROLL = 8

MAIN = NCH - (NCH % UNROLL)


def _make_sc_agg(with_counts: bool):
    mesh = plsc.VectorSubcoreMesh(core_axis_name="c", subcore_axis_name="s")

    out_type = [
        jax.ShapeDtypeStruct((N, D), jnp.float32),
        jax.ShapeDtypeStruct((N, D), jnp.float32),
    ]
    scratch = (
        [pltpu.VMEM((K,), jnp.int32) for _ in range(NI)]
        + [pltpu.VMEM((K,), jnp.int32) for _ in range(NI)]
        + [pltpu.VMEM((K, D), jnp.float32) for _ in range(R)]
        + [
            pltpu.VMEM((ZR, D), jnp.float32),
            pltpu.VMEM_SHARED((N, D), jnp.float32),
        ]
        + [pltpu.SemaphoreType.DMA for _ in range(NI)]
        + [pltpu.SemaphoreType.DMA for _ in range(R)]
        + [pltpu.SemaphoreType.DMA]
        + [pltpu.SemaphoreType.DMA]
    )
    if with_counts:
        out_type += [
            jax.ShapeDtypeStruct((N,), jnp.float32),
            jax.ShapeDtypeStruct((N,), jnp.float32),
        ]
        scratch += [
            pltpu.VMEM((K,), jnp.float32),
            pltpu.VMEM((CZ,), jnp.float32),
            pltpu.VMEM_SHARED((N,), jnp.float32),
        ]

    def body(x_hbm, src_hbm, dst_hbm, p0_hbm, p1_hbm, *rest):
        if with_counts:
            c0_hbm, c1_hbm, *rest = rest
        sidx = rest[:NI]
        didx = rest[NI:2 * NI]
        rows = rest[2 * NI:2 * NI + R]
        zbuf, acc = rest[2 * NI + R:2 * NI + R + 2]
        o = 2 * NI + R + 2
        semi = rest[o:o + NI]
        semg = rest[o + NI:o + NI + R]
        sems = rest[o + NI + R]
        semz = rest[o + NI + R + 1]
        if with_counts:
            ones, czbuf, cnt = rest[o + NI + R + 2:]
        cid = lax.axis_index("c")
        sid = lax.axis_index("s")
        wid = sid * NC + cid
        base = wid * EPW

        def issue_idx(ch, b):
            off = base + ch * K
            pltpu.async_copy(src_hbm.at[pl.ds(off, K)], sidx[b], semi[b])
            pltpu.async_copy(dst_hbm.at[pl.ds(off, K)], didx[b], semi[b])

        for ch0 in range(IL):
            issue_idx(ch0, ch0 % NI)

        def zfill(i, _):
            r = i // (D // 16)
            c = (i % (D // 16)) * 16
            zbuf[r, pl.ds(c, 16)] = jnp.zeros((16,), jnp.float32)
            return 0
        lax.fori_loop(0, ZR * (D // 16), zfill, 0)

        def zacc(j, _):
            pltpu.async_copy(zbuf, acc.at[pl.ds(sid * RPT + j * ZR, ZR)],
                             semz)
            return 0
        lax.fori_loop(0, (RPT + N - NS * RPT) // ZR, zacc, 0)

        if with_counts:
            for off in sorted({min(i * 16, K - 16)
                               for i in range((K + 15) // 16)}):
                ones[pl.ds(off, 16)] = jnp.ones((16,), jnp.float32)

            def czfill(i, _):
                czbuf[pl.ds(i * 16, 16)] = jnp.zeros((16,), jnp.float32)
                return 0
            lax.fori_loop(0, CZ // 16, czfill, 0)

            @pl.when(sid < N // CZ)
            def _():
                pltpu.sync_copy(czbuf, cnt.at[pl.ds(sid * CZ, CZ)])

        def zdrain(j, _):
            pltpu.make_async_copy(x_hbm.at[pl.ds(0, ZR)], zbuf, semz).wait()
            return 0
        lax.fori_loop(0, (RPT + N - NS * RPT) // ZR, zdrain, 0)

        plsc.subcore_barrier()

        def drain_idx(b):
            pltpu.make_async_copy(src_hbm.at[pl.ds(0, K)], sidx[b],
                                  semi[b]).wait()
            pltpu.make_async_copy(src_hbm.at[pl.ds(0, K)], didx[b],
                                  semi[b]).wait()

        def issue_gather(bi_, br_):
            pltpu.async_copy(x_hbm.at[sidx[bi_]], rows[br_], semg[br_])

        def drain_rows(sem_arr, b):
            pltpu.make_async_copy(x_hbm.at[pl.ds(0, K)], rows[b],
                                  sem_arr[b]).wait()

        def drain_scatter(b):
            pltpu.make_async_copy(x_hbm.at[pl.ds(0, K)], rows[b % R],
                                  sems).wait()

        def issue_scatter(ch, b):
            if isinstance(ch, int):
                if ch > 0:
                    drain_scatter(b)
            else:
                @pl.when(ch > 0)
                def _():
                    drain_scatter(b)
            pltpu.async_copy(rows[b % R], acc.at[didx[b % NI]], sems,
                             add=True)
            if with_counts:
                pltpu.sync_copy(ones, cnt.at[didx[b % NI]], add=True)

        def step(ch, b):
            bi = (b + IL) % NI
            bg_i = (b + GL) % NI
            bg = (b + GL) % R
            br = b % R

            @pl.when(ch + IL < NCH)
            def _():
                issue_idx(ch + IL, bi)

            @pl.when(ch + GL < NCH)
            def _():
                drain_idx(bg_i)
                issue_gather(bg_i, bg)

            drain_rows(semg, br)
            issue_scatter(ch, b)

        for ch in range(GL):
            drain_idx(ch % NI)
            issue_gather(ch % NI, ch % R)

        def round_(j2, _):
            for b in range(UNROLL):
                step(j2 * UNROLL + b, b)
            return 0
        lax.fori_loop(0, MAIN // UNROLL, round_, 0)
        for ch in range(MAIN, NCH):
            step(ch, ch % UNROLL)

        drain_scatter(0)

        plsc.subcore_barrier()

        @pl.when(cid == 0)
        def _():
            pltpu.sync_copy(acc.at[pl.ds(sid * RPT, RPT)],
                            p0_hbm.at[pl.ds(sid * RPT, RPT)])

            @pl.when(sid == NS - 1)
            def _():
                pltpu.sync_copy(acc.at[pl.ds(NS * RPT, N - NS * RPT)],
                                p0_hbm.at[pl.ds(NS * RPT, N - NS * RPT)])

        @pl.when(cid == 1)
        def _():
            pltpu.sync_copy(acc.at[pl.ds(sid * RPT, RPT)],
                            p1_hbm.at[pl.ds(sid * RPT, RPT)])

            @pl.when(sid == NS - 1)
            def _():
                pltpu.sync_copy(acc.at[pl.ds(NS * RPT, N - NS * RPT)],
                                p1_hbm.at[pl.ds(NS * RPT, N - NS * RPT)])

        if with_counts:
            @pl.when((cid == 0) & (sid < N // CZ))
            def _():
                pltpu.sync_copy(cnt.at[pl.ds(sid * CZ, CZ)], czbuf)
                pltpu.sync_copy(czbuf, c0_hbm.at[pl.ds(sid * CZ, CZ)])

            @pl.when((cid == 1) & (sid < N // CZ))
            def _():
                pltpu.sync_copy(cnt.at[pl.ds(sid * CZ, CZ)], czbuf)
                pltpu.sync_copy(czbuf, c1_hbm.at[pl.ds(sid * CZ, CZ)])

    return pl.kernel(body, out_type=out_type, mesh=mesh, scratch_types=scratch)


_sc_agg_counts = _make_sc_agg(True)
_sc_agg = _make_sc_agg(False)


BLK = 2000


def _tc_body(act, p0, p1, c0, c1, x, wn, ws, b, out):
    c = c0[...] + c1[...]
    s = p0[...] + p1[...]
    mean = s / jnp.maximum(c, 1.0)
    r = (jnp.dot(mean, wn[...], preferred_element_type=jnp.float32)
         + jnp.dot(x[...], ws[...], preferred_element_type=jnp.float32)
         + b[...])
    if act == "relu":
        out[...] = jnp.maximum(r, 0.0)
    else:
        m = jnp.max(r, axis=1, keepdims=True)
        lse = jnp.log(jnp.sum(jnp.exp(r - m), axis=1, keepdims=True)) + m
        out[...] = r - lse


def _tc_layer(p0, p1, c0, c1, x, wn, ws, b, act):
    return pl.pallas_call(
        functools.partial(_tc_body, act),
        grid=(N // BLK,),
        in_specs=[
            pl.BlockSpec((BLK, D), lambda i: (i, 0)),
            pl.BlockSpec((BLK, D), lambda i: (i, 0)),
            pl.BlockSpec((BLK, 1), lambda i: (i, 0)),
            pl.BlockSpec((BLK, 1), lambda i: (i, 0)),
            pl.BlockSpec((BLK, D), lambda i: (i, 0)),
            pl.BlockSpec((D, D), lambda i: (0, 0)),
            pl.BlockSpec((D, D), lambda i: (0, 0)),
            pl.BlockSpec((1, D), lambda i: (0, 0)),
        ],
        out_specs=pl.BlockSpec((BLK, D), lambda i: (i, 0)),
        out_shape=jax.ShapeDtypeStruct((N, D), jnp.float32),
    )(p0, p1, c0, c1, x, wn, ws, b.reshape(1, D))


def kernel(x, edge_index, W_self1, W_neigh1, b1, W_self2, W_neigh2, b2):
    src = edge_index[0].astype(jnp.int32)
    dst = edge_index[1].astype(jnp.int32)

    p0, p1, c0, c1 = _sc_agg_counts(x, src, dst)
    c0 = c0.reshape(N, 1)
    c1 = c1.reshape(N, 1)
    h = _tc_layer(p0, p1, c0, c1, x, W_neigh1, W_self1, b1, "relu")
    q0, q1 = _sc_agg(h, src, dst)
    return _tc_layer(q0, q1, c0, c1, h, W_neigh2, W_self2, b2, "logsoftmax")


# speedup vs baseline: 1.0099x; 1.0099x over previous
import functools

import jax
import jax.numpy as jnp
from jax import lax
from jax.experimental import pallas as pl
from jax.experimental.pallas import tpu as pltpu
from jax.experimental.pallas import tpu_sc as plsc

N = 10000
E = 320000
D = 128

NC, NS = 2, 16
NW = NC * NS
EPW = E // NW
K = 80
NCH = EPW // K
R = 4
NI = 8
IL = 6
GL = 2
RPT = 624
ZR = 16
CZ = 2000
UNROLL = 8

MAIN = NCH - (NCH % UNROLL)


def _make_sc_agg(with_counts: bool):
    mesh = plsc.VectorSubcoreMesh(core_axis_name="c", subcore_axis_name="s")

    out_type = [
        jax.ShapeDtypeStruct((N, D), jnp.float32),
        jax.ShapeDtypeStruct((N, D), jnp.float32),
    ]
    scratch = (
        [pltpu.VMEM((K,), jnp.int32) for _ in range(NI)]
        + [pltpu.VMEM((K,), jnp.int32) for _ in range(NI)]
        + [pltpu.VMEM((K, D), jnp.float32) for _ in range(R)]
        + [
            pltpu.VMEM((ZR, D), jnp.float32),
            pltpu.VMEM_SHARED((N, D), jnp.float32),
        ]
        + [pltpu.SemaphoreType.DMA for _ in range(NI)]
        + [pltpu.SemaphoreType.DMA for _ in range(R)]
        + [pltpu.SemaphoreType.DMA]
        + [pltpu.SemaphoreType.DMA]
    )
    if with_counts:
        out_type += [
            jax.ShapeDtypeStruct((N,), jnp.float32),
            jax.ShapeDtypeStruct((N,), jnp.float32),
        ]
        scratch += [
            pltpu.VMEM((K,), jnp.float32),
            pltpu.VMEM((CZ,), jnp.float32),
            pltpu.VMEM_SHARED((N,), jnp.float32),
        ]

    def body(x_hbm, src_hbm, dst_hbm, p0_hbm, p1_hbm, *rest):
        if with_counts:
            c0_hbm, c1_hbm, *rest = rest
        sidx = rest[:NI]
        didx = rest[NI:2 * NI]
        rows = rest[2 * NI:2 * NI + R]
        zbuf, acc = rest[2 * NI + R:2 * NI + R + 2]
        o = 2 * NI + R + 2
        semi = rest[o:o + NI]
        semg = rest[o + NI:o + NI + R]
        sems = rest[o + NI + R]
        semz = rest[o + NI + R + 1]
        if with_counts:
            ones, czbuf, cnt = rest[o + NI + R + 2:]
        cid = lax.axis_index("c")
        sid = lax.axis_index("s")
        wid = sid * NC + cid
        base = wid * EPW

        def issue_idx(ch, b):
            off = base + ch * K
            pltpu.async_copy(src_hbm.at[pl.ds(off, K)], sidx[b], semi[b])
            pltpu.async_copy(dst_hbm.at[pl.ds(off, K)], didx[b], semi[b])

        for ch0 in range(IL):
            issue_idx(ch0, ch0 % NI)

        def zfill(i, _):
            r = i // (D // 16)
            c = (i % (D // 16)) * 16
            zbuf[r, pl.ds(c, 16)] = jnp.zeros((16,), jnp.float32)
            return 0
        lax.fori_loop(0, ZR * (D // 16), zfill, 0)

        def zacc(j, _):
            pltpu.async_copy(zbuf, acc.at[pl.ds(sid * RPT + j * ZR, ZR)],
                             semz)
            return 0
        lax.fori_loop(0, (RPT + N - NS * RPT) // ZR, zacc, 0)

        if with_counts:
            for off in sorted({min(i * 16, K - 16)
                               for i in range((K + 15) // 16)}):
                ones[pl.ds(off, 16)] = jnp.ones((16,), jnp.float32)

            def czfill(i, _):
                czbuf[pl.ds(i * 16, 16)] = jnp.zeros((16,), jnp.float32)
                return 0
            lax.fori_loop(0, CZ // 16, czfill, 0)

            @pl.when(sid < N // CZ)
            def _():
                pltpu.sync_copy(czbuf, cnt.at[pl.ds(sid * CZ, CZ)])

        def zdrain(j, _):
            pltpu.make_async_copy(x_hbm.at[pl.ds(0, ZR)], zbuf, semz).wait()
            return 0
        lax.fori_loop(0, (RPT + N - NS * RPT) // ZR, zdrain, 0)

        plsc.subcore_barrier()

        def drain_idx(b):
            pltpu.make_async_copy(src_hbm.at[pl.ds(0, K)], sidx[b],
                                  semi[b]).wait()
            pltpu.make_async_copy(src_hbm.at[pl.ds(0, K)], didx[b],
                                  semi[b]).wait()

        def issue_gather(bi_, br_):
            pltpu.async_copy(x_hbm.at[sidx[bi_]], rows[br_], semg[br_])

        def drain_rows(sem_arr, b):
            pltpu.make_async_copy(x_hbm.at[pl.ds(0, K)], rows[b],
                                  sem_arr[b]).wait()

        def drain_scatter(b):
            pltpu.make_async_copy(x_hbm.at[pl.ds(0, K)], rows[b % R],
                                  sems).wait()

        def issue_scatter(ch, b):
            if isinstance(ch, int):
                if ch > 0:
                    drain_scatter(b)
            else:
                @pl.when(ch > 0)
                def _():
                    drain_scatter(b)
            pltpu.async_copy(rows[b % R], acc.at[didx[b % NI]], sems,
                             add=True)
            if with_counts:
                pltpu.sync_copy(ones, cnt.at[didx[b % NI]], add=True)

        def step(ch, b):
            bi = (b + IL) % NI
            bg_i = (b + GL) % NI
            bg = (b + GL) % R
            br = b % R

            @pl.when(ch + IL < NCH)
            def _():
                issue_idx(ch + IL, bi)

            @pl.when(ch + GL < NCH)
            def _():
                drain_idx(bg_i)
                issue_gather(bg_i, bg)

            drain_rows(semg, br)
            issue_scatter(ch, b)

        for ch in range(GL):
            drain_idx(ch % NI)
            issue_gather(ch % NI, ch % R)

        def round_(j2, _):
            for b in range(UNROLL):
                step(j2 * UNROLL + b, b)
            return 0
        lax.fori_loop(0, MAIN // UNROLL, round_, 0)
        for ch in range(MAIN, NCH):
            step(ch, ch % UNROLL)

        drain_scatter(0)

        plsc.subcore_barrier()

        @pl.when(cid == 0)
        def _():
            pltpu.sync_copy(acc.at[pl.ds(sid * RPT, RPT)],
                            p0_hbm.at[pl.ds(sid * RPT, RPT)])

            @pl.when(sid == NS - 1)
            def _():
                pltpu.sync_copy(acc.at[pl.ds(NS * RPT, N - NS * RPT)],
                                p0_hbm.at[pl.ds(NS * RPT, N - NS * RPT)])

        @pl.when(cid == 1)
        def _():
            pltpu.sync_copy(acc.at[pl.ds(sid * RPT, RPT)],
                            p1_hbm.at[pl.ds(sid * RPT, RPT)])

            @pl.when(sid == NS - 1)
            def _():
                pltpu.sync_copy(acc.at[pl.ds(NS * RPT, N - NS * RPT)],
                                p1_hbm.at[pl.ds(NS * RPT, N - NS * RPT)])

        if with_counts:
            @pl.when((cid == 0) & (sid < N // CZ))
            def _():
                pltpu.sync_copy(cnt.at[pl.ds(sid * CZ, CZ)], czbuf)
                pltpu.sync_copy(czbuf, c0_hbm.at[pl.ds(sid * CZ, CZ)])

            @pl.when((cid == 1) & (sid < N // CZ))
            def _():
                pltpu.sync_copy(cnt.at[pl.ds(sid * CZ, CZ)], czbuf)
                pltpu.sync_copy(czbuf, c1_hbm.at[pl.ds(sid * CZ, CZ)])

    return pl.kernel(body, out_type=out_type, mesh=mesh, scratch_types=scratch)


_sc_agg_counts = _make_sc_agg(True)
_sc_agg = _make_sc_agg(False)


BLK = 5000


def _tc_body(act, p0, p1, c0, c1, x, wn, ws, b, out):
    c = c0[...] + c1[...]
    s = p0[...] + p1[...]
    mean = s / jnp.maximum(c, 1.0)
    r = (jnp.dot(mean, wn[...], preferred_element_type=jnp.float32)
         + jnp.dot(x[...], ws[...], preferred_element_type=jnp.float32)
         + b[...])
    if act == "relu":
        out[...] = jnp.maximum(r, 0.0)
    else:
        m = jnp.max(r, axis=1, keepdims=True)
        lse = jnp.log(jnp.sum(jnp.exp(r - m), axis=1, keepdims=True)) + m
        out[...] = r - lse


def _tc_layer(p0, p1, c0, c1, x, wn, ws, b, act):
    return pl.pallas_call(
        functools.partial(_tc_body, act),
        grid=(N // BLK,),
        in_specs=[
            pl.BlockSpec((BLK, D), lambda i: (i, 0)),
            pl.BlockSpec((BLK, D), lambda i: (i, 0)),
            pl.BlockSpec((BLK, 1), lambda i: (i, 0)),
            pl.BlockSpec((BLK, 1), lambda i: (i, 0)),
            pl.BlockSpec((BLK, D), lambda i: (i, 0)),
            pl.BlockSpec((D, D), lambda i: (0, 0)),
            pl.BlockSpec((D, D), lambda i: (0, 0)),
            pl.BlockSpec((1, D), lambda i: (0, 0)),
        ],
        out_specs=pl.BlockSpec((BLK, D), lambda i: (i, 0)),
        out_shape=jax.ShapeDtypeStruct((N, D), jnp.float32),
    )(p0, p1, c0, c1, x, wn, ws, b.reshape(1, D))


def kernel(x, edge_index, W_self1, W_neigh1, b1, W_self2, W_neigh2, b2):
    src = edge_index[0].astype(jnp.int32)
    dst = edge_index[1].astype(jnp.int32)

    p0, p1, c0, c1 = _sc_agg_counts(x, src, dst)
    c0 = c0.reshape(N, 1)
    c1 = c1.reshape(N, 1)
    h = _tc_layer(p0, p1, c0, c1, x, W_neigh1, W_self1, b1, "relu")
    q0, q1 = _sc_agg(h, src, dst)
    return _tc_layer(q0, q1, c0, c1, h, W_neigh2, W_self2, b2, "logsoftmax")
